# baseline (speedup 1.0000x reference)
import functools

import jax
import jax.numpy as jnp
from jax import lax
from jax.experimental import pallas as pl
from jax.experimental.pallas import tpu as pltpu
from jax.experimental.pallas import tpu_sc as plsc

NC = 2
NS = 16
NW = NC * NS
CH = 128
NBUF = 2
RB = 2048
CB = 2176
V2P = 520064


def _build_table_tc(embT):
    d, v = embT.shape
    n_blk = V2P // CB

    def body(a_ref, b_ref, o_ref):
        o_ref[:, :d] = lax.transpose(a_ref[...], (1, 0))
        o_ref[:, d:] = lax.transpose(b_ref[...], (1, 0))

    return pl.pallas_call(
        body,
        grid=(n_blk,),
        in_specs=[
            pl.BlockSpec((d, CB), lambda i: (0, i)),
            pl.BlockSpec((d, CB), lambda i: (0, i + n_blk)),
        ],
        out_specs=pl.BlockSpec((CB, 2 * d), lambda i: (i, 0)),
        out_shape=jax.ShapeDtypeStruct((V2P, 2 * d), jnp.float32),
        compiler_params=pltpu.CompilerParams(
            dimension_semantics=("arbitrary",)
        ),
    )(embT, embT)


def _gather_sc(tblP, idx):
    r_total = idx.shape[0]
    d2 = tblP.shape[1]
    rows_per_w = r_total // NW
    n_ch = rows_per_w // CH
    idx3 = idx.reshape(NW, n_ch, CH)
    mesh = plsc.VectorSubcoreMesh(
        core_axis_name="c", subcore_axis_name="s", num_cores=NC, num_subcores=NS
    )

    @functools.partial(
        pl.kernel,
        mesh=mesh,
        out_type=jax.ShapeDtypeStruct((r_total, d2), jnp.float32),
        scratch_types=[
            pltpu.VMEM((n_ch, CH), jnp.int32),
            pltpu.VMEM((NBUF, CH, d2), jnp.float32),
            pltpu.SemaphoreType.DMA((NBUF,)),
        ],
        compiler_params=pltpu.CompilerParams(use_tc_tiling_on_sc=False),
    )
    def gather_kernel(table_hbm, idx_hbm, out_hbm, idx_v, rows_v, sems):
        wid = lax.axis_index("s") * NC + lax.axis_index("c")
        base = wid * rows_per_w
        pltpu.sync_copy(idx_hbm.at[wid], idx_v)
        for b in range(NBUF):
            pltpu.async_copy(table_hbm.at[idx_v.at[b]], rows_v.at[b], sems.at[b])

        @pl.loop(0, n_ch, step=NBUF)
        def _(j0):
            for b in range(NBUF):
                j = j0 + b
                pltpu.make_async_copy(
                    table_hbm.at[idx_v.at[j]], rows_v.at[b], sems.at[b]
                ).wait()
                pltpu.sync_copy(
                    rows_v.at[b], out_hbm.at[pl.ds(base + j * CH, CH)]
                )

                @pl.when(j + NBUF < n_ch)
                def _():
                    pltpu.async_copy(
                        table_hbm.at[idx_v.at[j + NBUF]], rows_v.at[b], sems.at[b]
                    )

    return gather_kernel(tblP, idx3)


def _project_tc(g2, idx3, bias3, proj_w, r_total, blk_per_field,
                field0, blk0, prev=None):
    r_slice = g2.shape[0]
    d = bias3.shape[2]
    h = proj_w.shape[0]
    n_blk = r_slice // RB

    def body(g_ref, i_ref, b_ref, w_ref, *rest):
        o_ref = rest[-1]
        hb = (i_ref[0] >= V2P).astype(jnp.int32)
        part = lax.transpose(hb, (1, 0))
        sel = jnp.where(part == 1, g_ref[:, d:], g_ref[:, :d])
        gb = (sel + b_ref[0]).astype(jnp.bfloat16)
        o_ref[...] = lax.dot_general(
            gb,
            w_ref[...].astype(jnp.bfloat16),
            (((1,), (1,)), ((), ())),
            preferred_element_type=jnp.float32,
        )

    in_specs = [
        pl.BlockSpec((RB, 2 * d), lambda i: (i, 0)),
        pl.BlockSpec((1, 1, RB), lambda i: (i, 0, 0)),
        pl.BlockSpec((1, 1, d), lambda i: (field0 + i // blk_per_field, 0, 0)),
        pl.BlockSpec((h, d), lambda i: (0, 0)),
    ]
    args = [g2, idx3, bias3, proj_w]
    aliases = {}
    if prev is not None:
        in_specs.append(pl.BlockSpec(memory_space=pl.ANY))
        args.append(prev)
        aliases = {4: 0}

    return pl.pallas_call(
        body,
        grid=(n_blk,),
        in_specs=in_specs,
        out_specs=pl.BlockSpec((RB, h), lambda i: (i + blk0, 0)),
        out_shape=jax.ShapeDtypeStruct((r_total, h), jnp.float32),
        input_output_aliases=aliases,
        compiler_params=pltpu.CompilerParams(
            dimension_semantics=("arbitrary",)
        ),
    )(*args)


def kernel(x_cat, category_offsets, embeddings, bias, proj_w):
    batch, n_fields = x_cat.shape
    h = proj_w.shape[0]
    d = bias.shape[1]
    tblP = _build_table_tc(embeddings.T)
    idx = (x_cat.T + category_offsets[:, None]).reshape(-1)
    r_total = idx.shape[0]
    gidx = jnp.where(idx >= V2P, idx - V2P, idx)
    half = r_total // 2
    f_half = n_fields // 2
    blk_per_field = batch // RB
    bias3 = bias.reshape(n_fields, 1, d)
    idx_a, idx_b = idx[:half], idx[half:]
    g2a = _gather_sc(tblP, gidx[:half])
    out_a = _project_tc(
        g2a, idx_a.reshape(half // RB, 1, RB), bias3, proj_w,
        r_total, blk_per_field, 0, 0,
    )
    g2b = _gather_sc(tblP, gidx[half:])
    out2 = _project_tc(
        g2b, idx_b.reshape(half // RB, 1, RB), bias3, proj_w,
        r_total, blk_per_field, f_half, half // RB, prev=out_a,
    )
    return out2.reshape(n_fields, batch, h).transpose(1, 0, 2)


# speedup vs baseline: 1.1064x; 1.1064x over previous
import functools

import jax
import jax.numpy as jnp
from jax import lax
from jax.experimental import pallas as pl
from jax.experimental.pallas import tpu as pltpu
from jax.experimental.pallas import tpu_sc as plsc

NC = 2
NS = 16
NW = NC * NS
CH = 128
NBUF = 2
RB = 2048
CB = 8192
V2P = 524288


def _build_table_tc(embT):
    d, v = embT.shape
    n_blk = V2P // CB
    last_b = (v - 1) // CB

    def body(a_ref, b_ref, o_ref):
        o_ref[:, :d] = lax.transpose(a_ref[...], (1, 0))
        o_ref[:, d:] = lax.transpose(b_ref[...], (1, 0))

    return pl.pallas_call(
        body,
        grid=(n_blk,),
        in_specs=[
            pl.BlockSpec((d, CB), lambda i: (0, i)),
            pl.BlockSpec((d, CB), lambda i: (0, jnp.minimum(i + n_blk, last_b))),
        ],
        out_specs=pl.BlockSpec((CB, 2 * d), lambda i: (i, 0)),
        out_shape=jax.ShapeDtypeStruct((V2P, 2 * d), jnp.float32),
        compiler_params=pltpu.CompilerParams(
            dimension_semantics=("arbitrary",)
        ),
    )(embT, embT)


def _gather_sc(tblP, gidx):
    r_total = gidx.shape[0]
    d2 = tblP.shape[1]
    rows_per_w = r_total // NW
    n_ch = rows_per_w // CH
    idx3 = gidx.reshape(NW, n_ch, CH)
    mesh = plsc.VectorSubcoreMesh(
        core_axis_name="c", subcore_axis_name="s", num_cores=NC, num_subcores=NS
    )

    @functools.partial(
        pl.kernel,
        mesh=mesh,
        out_type=jax.ShapeDtypeStruct((r_total, d2), jnp.float32),
        scratch_types=[
            pltpu.VMEM((n_ch, CH), jnp.int32),
            pltpu.VMEM((NBUF, CH, d2), jnp.float32),
            pltpu.SemaphoreType.DMA((NBUF,)),
        ],
        compiler_params=pltpu.CompilerParams(use_tc_tiling_on_sc=False),
    )
    def gather_kernel(table_hbm, idx_hbm, out_hbm, idx_v, rows_v, sems):
        wid = lax.axis_index("s") * NC + lax.axis_index("c")
        base = wid * rows_per_w
        pltpu.sync_copy(idx_hbm.at[wid], idx_v)
        for b in range(NBUF):
            pltpu.async_copy(table_hbm.at[idx_v.at[b]], rows_v.at[b], sems.at[b])

        @pl.loop(0, n_ch, step=NBUF)
        def _(j0):
            for b in range(NBUF):
                j = j0 + b
                pltpu.make_async_copy(
                    table_hbm.at[idx_v.at[j]], rows_v.at[b], sems.at[b]
                ).wait()
                pltpu.sync_copy(
                    rows_v.at[b], out_hbm.at[pl.ds(base + j * CH, CH)]
                )

                @pl.when(j + NBUF < n_ch)
                def _():
                    pltpu.async_copy(
                        table_hbm.at[idx_v.at[j + NBUF]], rows_v.at[b], sems.at[b]
                    )

    return gather_kernel(tblP, idx3)


def _project_tc(g2, idx3, bias, proj_w, rows_per_field):
    r_total = g2.shape[0]
    d = bias.shape[1]
    h = proj_w.shape[0]
    n_blk = r_total // RB
    blk_per_field = rows_per_field // RB

    def body(g_ref, i_ref, b_ref, w_ref, o_ref):
        hb = (i_ref[0] >= V2P).astype(jnp.int32)
        part = lax.transpose(hb, (1, 0))
        sel = jnp.where(part == 1, g_ref[:, d:], g_ref[:, :d])
        gb = (sel + b_ref[0]).astype(jnp.bfloat16)
        o_ref[...] = lax.dot_general(
            gb,
            w_ref[...].astype(jnp.bfloat16),
            (((1,), (1,)), ((), ())),
            preferred_element_type=jnp.float32,
        )

    return pl.pallas_call(
        body,
        grid=(n_blk,),
        in_specs=[
            pl.BlockSpec((RB, 2 * d), lambda i: (i, 0)),
            pl.BlockSpec((1, 1, RB), lambda i: (i, 0, 0)),
            pl.BlockSpec((1, 1, d), lambda i: (i // blk_per_field, 0, 0)),
            pl.BlockSpec((h, d), lambda i: (0, 0)),
        ],
        out_specs=pl.BlockSpec((RB, h), lambda i: (i, 0)),
        out_shape=jax.ShapeDtypeStruct((r_total, h), jnp.float32),
        compiler_params=pltpu.CompilerParams(
            dimension_semantics=("arbitrary",)
        ),
    )(g2, idx3, bias.reshape(bias.shape[0], 1, d), proj_w)


def kernel(x_cat, category_offsets, embeddings, bias, proj_w):
    batch, n_fields = x_cat.shape
    h = proj_w.shape[0]
    tblP = _build_table_tc(embeddings.T)
    idx = (x_cat.T + category_offsets[:, None]).reshape(-1)
    r_total = idx.shape[0]
    g2 = _gather_sc(tblP, idx & (V2P - 1))
    out2 = _project_tc(
        g2, idx.reshape(r_total // RB, 1, RB), bias, proj_w, batch
    )
    return out2.reshape(n_fields, batch, h).transpose(1, 0, 2)


# speedup vs baseline: 1.1292x; 1.0205x over previous
import functools

import jax
import jax.numpy as jnp
from jax import lax
from jax.experimental import pallas as pl
from jax.experimental.pallas import tpu as pltpu
from jax.experimental.pallas import tpu_sc as plsc

NC = 2
NS = 16
NW = NC * NS
CH = 128
NBUF = 4
RB = 2048
CB = 16384
V2P = 524288


def _build_table_tc(embT):
    d, v = embT.shape
    n_blk = V2P // CB
    last_b = (v - 1) // CB

    def body(a_ref, b_ref, o_ref):
        o_ref[:, :d] = lax.transpose(a_ref[...], (1, 0))
        o_ref[:, d:] = lax.transpose(b_ref[...], (1, 0))

    return pl.pallas_call(
        body,
        grid=(n_blk,),
        in_specs=[
            pl.BlockSpec((d, CB), lambda i: (0, i)),
            pl.BlockSpec((d, CB), lambda i: (0, jnp.minimum(i + n_blk, last_b))),
        ],
        out_specs=pl.BlockSpec((CB, 2 * d), lambda i: (i, 0)),
        out_shape=jax.ShapeDtypeStruct((V2P, 2 * d), jnp.float32),
        compiler_params=pltpu.CompilerParams(
            dimension_semantics=("arbitrary",)
        ),
    )(embT, embT)


def _gather_sc(tblP, gidx):
    r_total = gidx.shape[0]
    d2 = tblP.shape[1]
    rows_per_w = r_total // NW
    n_ch = rows_per_w // CH
    idx3 = gidx.reshape(NW, n_ch, CH)
    mesh = plsc.VectorSubcoreMesh(
        core_axis_name="c", subcore_axis_name="s", num_cores=NC, num_subcores=NS
    )

    @functools.partial(
        pl.kernel,
        mesh=mesh,
        out_type=jax.ShapeDtypeStruct((r_total, d2), jnp.float32),
        scratch_types=[
            pltpu.VMEM((n_ch, CH), jnp.int32),
            pltpu.VMEM((NBUF, CH, d2), jnp.float32),
            pltpu.SemaphoreType.DMA((NBUF,)),
        ],
        compiler_params=pltpu.CompilerParams(use_tc_tiling_on_sc=False),
    )
    def gather_kernel(table_hbm, idx_hbm, out_hbm, idx_v, rows_v, sems):
        wid = lax.axis_index("s") * NC + lax.axis_index("c")
        base = wid * rows_per_w
        pltpu.sync_copy(idx_hbm.at[wid], idx_v)
        for b in range(NBUF):
            pltpu.async_copy(table_hbm.at[idx_v.at[b]], rows_v.at[b], sems.at[b])

        @pl.loop(0, n_ch, step=NBUF)
        def _(j0):
            for b in range(NBUF):
                j = j0 + b
                pltpu.make_async_copy(
                    table_hbm.at[idx_v.at[j]], rows_v.at[b], sems.at[b]
                ).wait()
                pltpu.sync_copy(
                    rows_v.at[b], out_hbm.at[pl.ds(base + j * CH, CH)]
                )

                @pl.when(j + NBUF < n_ch)
                def _():
                    pltpu.async_copy(
                        table_hbm.at[idx_v.at[j + NBUF]], rows_v.at[b], sems.at[b]
                    )

    return gather_kernel(tblP, idx3)


def _project_tc(g2, idx3, bias, proj_w, rows_per_field):
    r_total = g2.shape[0]
    d = bias.shape[1]
    h = proj_w.shape[0]
    n_blk = r_total // RB
    blk_per_field = rows_per_field // RB

    def body(g_ref, i_ref, b_ref, w_ref, o_ref):
        hb = (i_ref[0] >= V2P).astype(jnp.int32)
        part = lax.transpose(hb, (1, 0))
        sel = jnp.where(part == 1, g_ref[:, d:], g_ref[:, :d])
        gb = (sel + b_ref[0]).astype(jnp.bfloat16)
        o_ref[...] = lax.dot_general(
            gb,
            w_ref[...].astype(jnp.bfloat16),
            (((1,), (1,)), ((), ())),
            preferred_element_type=jnp.float32,
        )

    return pl.pallas_call(
        body,
        grid=(n_blk,),
        in_specs=[
            pl.BlockSpec((RB, 2 * d), lambda i: (i, 0)),
            pl.BlockSpec((1, 1, RB), lambda i: (i, 0, 0)),
            pl.BlockSpec((1, 1, d), lambda i: (i // blk_per_field, 0, 0)),
            pl.BlockSpec((h, d), lambda i: (0, 0)),
        ],
        out_specs=pl.BlockSpec((RB, h), lambda i: (i, 0)),
        out_shape=jax.ShapeDtypeStruct((r_total, h), jnp.float32),
        compiler_params=pltpu.CompilerParams(
            dimension_semantics=("arbitrary",)
        ),
    )(g2, idx3, bias.reshape(bias.shape[0], 1, d), proj_w)


def kernel(x_cat, category_offsets, embeddings, bias, proj_w):
    batch, n_fields = x_cat.shape
    h = proj_w.shape[0]
    tblP = _build_table_tc(embeddings.T)
    idx = (x_cat.T + category_offsets[:, None]).reshape(-1)
    r_total = idx.shape[0]
    g2 = _gather_sc(tblP, idx & (V2P - 1))
    out2 = _project_tc(
        g2, idx.reshape(r_total // RB, 1, RB), bias, proj_w, batch
    )
    return out2.reshape(n_fields, batch, h).transpose(1, 0, 2)


# speedup vs baseline: 1.1997x; 1.0625x over previous
import functools

import jax
import jax.numpy as jnp
from jax import lax
from jax.experimental import pallas as pl
from jax.experimental.pallas import tpu as pltpu
from jax.experimental.pallas import tpu_sc as plsc

NC = 2
NS = 16
NW = NC * NS
CH = 128
NBUF = 4
RB = 4096
CB = 16384
V2P = 524288


def _build_table_tc(embT):
    d, v = embT.shape
    n_blk = V2P // CB
    last_b = (v - 1) // CB

    def body(a_ref, b_ref, o_ref):
        o_ref[:, :d] = lax.transpose(a_ref[...], (1, 0))
        o_ref[:, d:] = lax.transpose(b_ref[...], (1, 0))

    return pl.pallas_call(
        body,
        grid=(n_blk,),
        in_specs=[
            pl.BlockSpec((d, CB), lambda i: (0, i)),
            pl.BlockSpec((d, CB), lambda i: (0, jnp.minimum(i + n_blk, last_b))),
        ],
        out_specs=pl.BlockSpec((CB, 2 * d), lambda i: (i, 0)),
        out_shape=jax.ShapeDtypeStruct((V2P, 2 * d), jnp.float32),
        compiler_params=pltpu.CompilerParams(
            dimension_semantics=("arbitrary",)
        ),
    )(embT, embT)


def _gather_sc(tblP, gidx):
    r_total = gidx.shape[0]
    d2 = tblP.shape[1]
    rows_per_w = r_total // NW
    n_ch = rows_per_w // CH
    idx3 = gidx.reshape(NW, n_ch, CH)
    mesh = plsc.VectorSubcoreMesh(
        core_axis_name="c", subcore_axis_name="s", num_cores=NC, num_subcores=NS
    )

    @functools.partial(
        pl.kernel,
        mesh=mesh,
        out_type=jax.ShapeDtypeStruct((r_total, d2), jnp.float32),
        scratch_types=[
            pltpu.VMEM((n_ch, CH), jnp.int32),
            pltpu.VMEM((NBUF, CH, d2), jnp.float32),
            pltpu.SemaphoreType.DMA((NBUF,)),
        ],
        compiler_params=pltpu.CompilerParams(use_tc_tiling_on_sc=False),
    )
    def gather_kernel(table_hbm, idx_hbm, out_hbm, idx_v, rows_v, sems):
        wid = lax.axis_index("s") * NC + lax.axis_index("c")
        base = wid * rows_per_w
        pltpu.sync_copy(idx_hbm.at[wid], idx_v)
        for b in range(NBUF):
            pltpu.async_copy(table_hbm.at[idx_v.at[b]], rows_v.at[b], sems.at[b])

        @pl.loop(0, n_ch, step=NBUF)
        def _(j0):
            for b in range(NBUF):
                j = j0 + b
                pltpu.make_async_copy(
                    table_hbm.at[idx_v.at[j]], rows_v.at[b], sems.at[b]
                ).wait()
                pltpu.sync_copy(
                    rows_v.at[b], out_hbm.at[pl.ds(base + j * CH, CH)]
                )

                @pl.when(j + NBUF < n_ch)
                def _():
                    pltpu.async_copy(
                        table_hbm.at[idx_v.at[j + NBUF]], rows_v.at[b], sems.at[b]
                    )

    return gather_kernel(tblP, idx3)


def _project_tc(g2, idx3, bias, proj_w, rows_per_field):
    r_total = g2.shape[0]
    d = bias.shape[1]
    h = proj_w.shape[0]
    n_blk = r_total // RB
    blk_per_field = rows_per_field // RB

    def body(g_ref, i_ref, b_ref, w_ref, o_ref):
        hb = (i_ref[0] >= V2P).astype(jnp.int32)
        part = lax.transpose(hb, (1, 0))
        sel = jnp.where(part == 1, g_ref[:, d:], g_ref[:, :d])
        gb = (sel + b_ref[0]).astype(jnp.bfloat16)
        o_ref[...] = lax.dot_general(
            gb,
            w_ref[...].astype(jnp.bfloat16),
            (((1,), (1,)), ((), ())),
            preferred_element_type=jnp.float32,
        )

    return pl.pallas_call(
        body,
        grid=(n_blk,),
        in_specs=[
            pl.BlockSpec((RB, 2 * d), lambda i: (i, 0)),
            pl.BlockSpec((1, 1, RB), lambda i: (i, 0, 0)),
            pl.BlockSpec((1, 1, d), lambda i: (i // blk_per_field, 0, 0)),
            pl.BlockSpec((h, d), lambda i: (0, 0)),
        ],
        out_specs=pl.BlockSpec((RB, h), lambda i: (i, 0)),
        out_shape=jax.ShapeDtypeStruct((r_total, h), jnp.float32),
        compiler_params=pltpu.CompilerParams(
            dimension_semantics=("arbitrary",)
        ),
    )(g2, idx3, bias.reshape(bias.shape[0], 1, d), proj_w)


def kernel(x_cat, category_offsets, embeddings, bias, proj_w):
    batch, n_fields = x_cat.shape
    h = proj_w.shape[0]
    tblP = _build_table_tc(embeddings.T)
    idx = (x_cat.T + category_offsets[:, None]).reshape(-1)
    r_total = idx.shape[0]
    g2 = _gather_sc(tblP, idx & (V2P - 1))
    out2 = _project_tc(
        g2, idx.reshape(r_total // RB, 1, RB), bias, proj_w, batch
    )
    return out2.reshape(n_fields, batch, h).transpose(1, 0, 2)
